# baseline (speedup 1.0000x reference)
import functools

import jax
import jax.numpy as jnp
from jax import lax
from jax.experimental import pallas as pl
from jax.experimental.pallas import tpu as pltpu

EPS = 1e-5
INV_SQRT2 = 0.7071067811865476
HC = 16


def _fold_bn(conv_bias, gamma, beta, mean, var):
    scale = gamma / jnp.sqrt(var + EPS)
    bias = beta + (conv_bias - mean) * scale
    return scale, bias


def _conv_rows(src, r0, wc, dil, wg, C, W):
    taps = [src[r0 + (ky - 1) * dil:r0 + (ky - 1) * dil + HC, :, :]
            .reshape(HC * wc, C) for ky in range(3)]
    xcol = jnp.concatenate(taps, axis=-1)
    u = jnp.dot(xcol, wg, preferred_element_type=jnp.float32)
    u3 = u.reshape(HC, wc, 3 * C)
    v = (u3[:, 8 - dil:8 - dil + W, 0:C]
         + u3[:, 8:8 + W, C:2 * C]
         + u3[:, 8 + dil:8 + dil + W, 2 * C:3 * C])
    return v.reshape(HC * W, C)


def _fused_kernel(xp_ref, wA_ref, bA_ref, w9_ref, bB_ref, w3_ref, b3_ref,
                  w1_ref, b1_ref, sb2_ref, o_ref, *, H, W, C, Ca):
    f32 = jnp.float32
    bf16 = jnp.bfloat16
    WF = W + 16

    x2 = xp_ref[0].reshape(H * W, Ca)
    yA = jnp.maximum(jnp.dot(x2, wA_ref[...], preferred_element_type=f32)
                     + bA_ref[...], 0.0)
    y_pad = jnp.pad(yA.astype(bf16).reshape(H, W, C),
                    ((9, 9), (8, 8), (0, 0)))

    chunks = []
    for h0 in range(0, H, HC):
        v = _conv_rows(y_pad, h0 + 9, WF, 1, w9_ref[...], C, W)
        z = jnp.maximum(v + bB_ref[...], 0.0)
        chunks.append(z.astype(bf16).reshape(HC, W, C))
    xb = jnp.pad(jnp.concatenate(chunks, axis=0),
                 ((8, 8), (8, 8), (0, 0)))

    for i0 in range(0, H, HC):
        acc = jnp.zeros((HC * W, C), f32)
        for bi, dil in enumerate((1, 3, 5)):
            v = _conv_rows(xb, i0 + 8, WF, dil, w3_ref[bi], C, W)
            zb = jnp.maximum(v + b3_ref[bi:bi + 1, :], 0.0)
            acc = acc + jnp.dot(zb.astype(bf16), w1_ref[bi],
                                preferred_element_type=f32)
        y = jnp.maximum(acc + b1_ref[...], 0.0)
        y = y + yA[i0 * W:(i0 + HC) * W, :]
        y = y * sb2_ref[0:1, :] + sb2_ref[1:2, :]
        y = 0.5 * y * (1.0 + lax.erf(y * INV_SQRT2))
        o_ref[0, i0 * W:(i0 + HC) * W, :] = y.astype(o_ref.dtype)


def _regroup(w, scale):
    C = w.shape[-1]
    return jnp.transpose(w * scale[None, None, None, :],
                         (0, 2, 1, 3)).reshape(3 * C, 3 * C)


def kernel(x, conv_w, conv_b, conv_bn_gamma, conv_bn_beta, conv_bn_mean,
           conv_bn_var, d3_w, d3_b, d3_bn_gamma, d3_bn_beta, d3_bn_mean,
           d3_bn_var, d1_w, d1_b, d1_bn_gamma, d1_bn_beta, d1_bn_mean,
           d1_bn_var, dd3_w, dd3_b, dd3_bn_gamma, dd3_bn_beta, dd3_bn_mean,
           dd3_bn_var, dd5_w, dd5_b, dd5_bn_gamma, dd5_bn_beta, dd5_bn_mean,
           dd5_bn_var, c1_w, c1_b, c1_bn_gamma, c1_bn_beta, c1_bn_mean,
           c1_bn_var, out_bn_gamma, out_bn_beta, out_bn_mean, out_bn_var):
    B, Cin, H, W = x.shape
    C = conv_b.shape[0]
    bf16 = jnp.bfloat16

    sA, bA = _fold_bn(conv_b, conv_bn_gamma, conv_bn_beta, conv_bn_mean,
                      conv_bn_var)
    sB, bB = _fold_bn(d3_b, d3_bn_gamma, d3_bn_beta, d3_bn_mean, d3_bn_var)
    s1d, b1d = _fold_bn(d1_b, d1_bn_gamma, d1_bn_beta, d1_bn_mean, d1_bn_var)
    s3d, b3d = _fold_bn(dd3_b, dd3_bn_gamma, dd3_bn_beta, dd3_bn_mean,
                        dd3_bn_var)
    s5d, b5d = _fold_bn(dd5_b, dd5_bn_gamma, dd5_bn_beta, dd5_bn_mean,
                        dd5_bn_var)
    s1, b1 = _fold_bn(c1_b, c1_bn_gamma, c1_bn_beta, c1_bn_mean, c1_bn_var)
    s2 = out_bn_gamma / jnp.sqrt(out_bn_var + EPS)
    b2 = out_bn_beta - out_bn_mean * s2

    wA = (conv_w * sA[None, :]).astype(bf16)
    w9 = _regroup(d3_w, sB).astype(bf16)
    w3 = jnp.stack([_regroup(d1_w, s1d), _regroup(dd3_w, s3d),
                    _regroup(dd5_w, s5d)]).astype(bf16)
    b3 = jnp.stack([b1d, b3d, b5d])
    w1 = (c1_w.reshape(3, C, C) * s1[None, None, :]).astype(bf16)
    sb2 = jnp.stack([s2, b2])

    return _forward(x, wA, bA.reshape(1, C), w9, bB.reshape(1, C), w3, b3,
                    w1, b1.reshape(1, C), sb2, H=H, W=W, C=C, Ca=Cin)


def _forward(x, wA, bA, w9, bB, w3, b3, w1, b1, sb2, *, H, W, C, Ca):
    B = x.shape[0]
    xp = jnp.transpose(x, (0, 2, 3, 1)).astype(jnp.bfloat16)
    kern = functools.partial(_fused_kernel, H=H, W=W, C=C, Ca=Ca)
    out = pl.pallas_call(
        kern,
        out_shape=jax.ShapeDtypeStruct((B, H * W, C), jnp.bfloat16),
        grid=(B,),
        in_specs=[
            pl.BlockSpec((1, H, W, Ca), lambda b: (b, 0, 0, 0)),
            pl.BlockSpec((Ca, C), lambda b: (0, 0)),
            pl.BlockSpec((1, C), lambda b: (0, 0)),
            pl.BlockSpec((3 * C, 3 * C), lambda b: (0, 0)),
            pl.BlockSpec((1, C), lambda b: (0, 0)),
            pl.BlockSpec((3, 3 * C, 3 * C), lambda b: (0, 0, 0)),
            pl.BlockSpec((3, C), lambda b: (0, 0)),
            pl.BlockSpec((3, C, C), lambda b: (0, 0, 0)),
            pl.BlockSpec((1, C), lambda b: (0, 0)),
            pl.BlockSpec((2, C), lambda b: (0, 0)),
        ],
        out_specs=pl.BlockSpec((1, H * W, C), lambda b: (b, 0, 0)),
        compiler_params=pltpu.CompilerParams(
            dimension_semantics=("parallel",),
            vmem_limit_bytes=60 * 1024 * 1024),
    )(xp, wA, bA, w9, bB, w3, b3, w1, b1, sb2)

    return jnp.transpose(out.reshape(B, H, W, C),
                         (0, 3, 1, 2)).astype(jnp.float32)


# speedup vs baseline: 1.0765x; 1.0765x over previous
import functools

import jax
import jax.numpy as jnp
from jax import lax
from jax.experimental import pallas as pl
from jax.experimental.pallas import tpu as pltpu

EPS = 1e-5
INV_SQRT2 = 0.7071067811865476
HC = 16


def _fold_bn(conv_bias, gamma, beta, mean, var):
    scale = gamma / jnp.sqrt(var + EPS)
    bias = beta + (conv_bias - mean) * scale
    return scale, bias


def _conv_rows(src, r0, wc, dil, wg, C, W):
    taps = [src[r0 + (ky - 1) * dil:r0 + (ky - 1) * dil + HC, :, :]
            .reshape(HC * wc, C) for ky in range(3)]
    xcol = jnp.concatenate(taps, axis=-1)
    u = jnp.dot(xcol, wg, preferred_element_type=jnp.float32)
    u3 = u.astype(jnp.bfloat16).reshape(HC, wc, 3 * C)
    v = (u3[:, 8 - dil:8 - dil + W, 0:C]
         + u3[:, 8:8 + W, C:2 * C]
         + u3[:, 8 + dil:8 + dil + W, 2 * C:3 * C])
    return v.reshape(HC * W, C)


def _fused_kernel(xp_ref, wA_ref, bA_ref, w9_ref, bB_ref, w3_ref, b3_ref,
                  w1_ref, b1_ref, sb2_ref, o_ref, *, H, W, C, Ca):
    f32 = jnp.float32
    bf16 = jnp.bfloat16
    WF = W + 16

    x2 = xp_ref[0].reshape(H * W, Ca)
    yA = jnp.maximum(jnp.dot(x2, wA_ref[...], preferred_element_type=f32)
                     + bA_ref[...], 0.0)
    y_pad = jnp.pad(yA.astype(bf16).reshape(H, W, C),
                    ((9, 9), (8, 8), (0, 0)))

    chunks = []
    bB = bB_ref[...].astype(bf16)
    for h0 in range(0, H, HC):
        v = _conv_rows(y_pad, h0 + 9, WF, 1, w9_ref[...], C, W)
        chunks.append(jnp.maximum(v + bB, 0.0).reshape(HC, W, C))
    xb = jnp.pad(jnp.concatenate(chunks, axis=0),
                 ((8, 8), (8, 8), (0, 0)))

    for i0 in range(0, H, HC):
        acc = jnp.zeros((HC * W, C), f32)
        for bi, dil in enumerate((1, 3, 5)):
            v = _conv_rows(xb, i0 + 8, WF, dil, w3_ref[bi], C, W)
            zb = jnp.maximum(v + b3_ref[bi:bi + 1, :].astype(bf16), 0.0)
            acc = acc + jnp.dot(zb, w1_ref[bi],
                                preferred_element_type=f32)
        y = jnp.maximum(acc + b1_ref[...], 0.0)
        y = y + yA[i0 * W:(i0 + HC) * W, :]
        y = y * sb2_ref[0:1, :] + sb2_ref[1:2, :]
        y = 0.5 * y * (1.0 + lax.erf(y * INV_SQRT2))
        o_ref[0, i0 * W:(i0 + HC) * W, :] = y.astype(o_ref.dtype)


def _regroup(w, scale):
    C = w.shape[-1]
    return jnp.transpose(w * scale[None, None, None, :],
                         (0, 2, 1, 3)).reshape(3 * C, 3 * C)


def kernel(x, conv_w, conv_b, conv_bn_gamma, conv_bn_beta, conv_bn_mean,
           conv_bn_var, d3_w, d3_b, d3_bn_gamma, d3_bn_beta, d3_bn_mean,
           d3_bn_var, d1_w, d1_b, d1_bn_gamma, d1_bn_beta, d1_bn_mean,
           d1_bn_var, dd3_w, dd3_b, dd3_bn_gamma, dd3_bn_beta, dd3_bn_mean,
           dd3_bn_var, dd5_w, dd5_b, dd5_bn_gamma, dd5_bn_beta, dd5_bn_mean,
           dd5_bn_var, c1_w, c1_b, c1_bn_gamma, c1_bn_beta, c1_bn_mean,
           c1_bn_var, out_bn_gamma, out_bn_beta, out_bn_mean, out_bn_var):
    B, Cin, H, W = x.shape
    C = conv_b.shape[0]
    bf16 = jnp.bfloat16

    sA, bA = _fold_bn(conv_b, conv_bn_gamma, conv_bn_beta, conv_bn_mean,
                      conv_bn_var)
    sB, bB = _fold_bn(d3_b, d3_bn_gamma, d3_bn_beta, d3_bn_mean, d3_bn_var)
    s1d, b1d = _fold_bn(d1_b, d1_bn_gamma, d1_bn_beta, d1_bn_mean, d1_bn_var)
    s3d, b3d = _fold_bn(dd3_b, dd3_bn_gamma, dd3_bn_beta, dd3_bn_mean,
                        dd3_bn_var)
    s5d, b5d = _fold_bn(dd5_b, dd5_bn_gamma, dd5_bn_beta, dd5_bn_mean,
                        dd5_bn_var)
    s1, b1 = _fold_bn(c1_b, c1_bn_gamma, c1_bn_beta, c1_bn_mean, c1_bn_var)
    s2 = out_bn_gamma / jnp.sqrt(out_bn_var + EPS)
    b2 = out_bn_beta - out_bn_mean * s2

    wA = (conv_w * sA[None, :]).astype(bf16)
    w9 = _regroup(d3_w, sB).astype(bf16)
    w3 = jnp.stack([_regroup(d1_w, s1d), _regroup(dd3_w, s3d),
                    _regroup(dd5_w, s5d)]).astype(bf16)
    b3 = jnp.stack([b1d, b3d, b5d])
    w1 = (c1_w.reshape(3, C, C) * s1[None, None, :]).astype(bf16)
    sb2 = jnp.stack([s2, b2])

    return _forward(x, wA, bA.reshape(1, C), w9, bB.reshape(1, C), w3, b3,
                    w1, b1.reshape(1, C), sb2, H=H, W=W, C=C, Ca=Cin)


def _forward(x, wA, bA, w9, bB, w3, b3, w1, b1, sb2, *, H, W, C, Ca):
    B = x.shape[0]
    xp = jnp.transpose(x, (0, 2, 3, 1)).astype(jnp.bfloat16)
    kern = functools.partial(_fused_kernel, H=H, W=W, C=C, Ca=Ca)
    out = pl.pallas_call(
        kern,
        out_shape=jax.ShapeDtypeStruct((B, H * W, C), jnp.float32),
        grid=(B,),
        in_specs=[
            pl.BlockSpec((1, H, W, Ca), lambda b: (b, 0, 0, 0)),
            pl.BlockSpec((Ca, C), lambda b: (0, 0)),
            pl.BlockSpec((1, C), lambda b: (0, 0)),
            pl.BlockSpec((3 * C, 3 * C), lambda b: (0, 0)),
            pl.BlockSpec((1, C), lambda b: (0, 0)),
            pl.BlockSpec((3, 3 * C, 3 * C), lambda b: (0, 0, 0)),
            pl.BlockSpec((3, C), lambda b: (0, 0)),
            pl.BlockSpec((3, C, C), lambda b: (0, 0, 0)),
            pl.BlockSpec((1, C), lambda b: (0, 0)),
            pl.BlockSpec((2, C), lambda b: (0, 0)),
        ],
        out_specs=pl.BlockSpec((1, H * W, C), lambda b: (b, 0, 0)),
        compiler_params=pltpu.CompilerParams(
            dimension_semantics=("parallel",),
            vmem_limit_bytes=60 * 1024 * 1024),
    )(xp, wA, bA, w9, bB, w3, b3, w1, b1, sb2)

    return jnp.transpose(out.reshape(B, H, W, C), (0, 3, 1, 2))


# speedup vs baseline: 1.1081x; 1.0294x over previous
import functools

import jax
import jax.numpy as jnp
from jax import lax
from jax.experimental import pallas as pl
from jax.experimental.pallas import tpu as pltpu

EPS = 1e-5
INV_SQRT2 = 0.7071067811865476
HC = 16


def _fold_bn(conv_bias, gamma, beta, mean, var):
    scale = gamma / jnp.sqrt(var + EPS)
    bias = beta + (conv_bias - mean) * scale
    return scale, bias


def _conv_rows(src, r0, wc, dil, wg, C, W):
    taps = [src[r0 + (ky - 1) * dil:r0 + (ky - 1) * dil + HC, :, :]
            .reshape(HC * wc, C) for ky in range(3)]
    xcol = jnp.concatenate(taps, axis=-1)
    u = jnp.dot(xcol, wg, preferred_element_type=jnp.float32)
    u3 = u.astype(jnp.bfloat16).reshape(HC, wc, 3 * C)
    v = (u3[:, 8 - dil:8 - dil + W, 0:C]
         + u3[:, 8:8 + W, C:2 * C]
         + u3[:, 8 + dil:8 + dil + W, 2 * C:3 * C])
    return v.reshape(HC * W, C)


def _fused_kernel(xp_ref, wA_ref, bA_ref, w9_ref, bB_ref, w3_ref, b3_ref,
                  w1_ref, b1_ref, sb2_ref, o_ref, *, H, W, C, Ca):
    f32 = jnp.float32
    bf16 = jnp.bfloat16
    WF = W + 16

    yA = lax.dot_general(xp_ref[0], wA_ref[...], (((0,), (0,)), ((), ())),
                         preferred_element_type=f32)
    yA = jnp.maximum(yA + bA_ref[...], 0.0)
    y_pad = jnp.pad(yA.astype(bf16).reshape(H, W, C),
                    ((9, 9), (8, 8), (0, 0)))

    chunks = []
    bB = bB_ref[...].astype(bf16)
    for h0 in range(0, H, HC):
        v = _conv_rows(y_pad, h0 + 9, WF, 1, w9_ref[...], C, W)
        chunks.append(jnp.maximum(v + bB, 0.0).reshape(HC, W, C))
    xb = jnp.pad(jnp.concatenate(chunks, axis=0),
                 ((8, 8), (8, 8), (0, 0)))

    for i0 in range(0, H, HC):
        acc = jnp.zeros((HC * W, C), f32)
        for bi, dil in enumerate((1, 3, 5)):
            v = _conv_rows(xb, i0 + 8, WF, dil, w3_ref[bi], C, W)
            zb = jnp.maximum(v + b3_ref[bi:bi + 1, :].astype(bf16), 0.0)
            acc = acc + jnp.dot(zb, w1_ref[bi],
                                preferred_element_type=f32)
        y = jnp.maximum(acc + b1_ref[...], 0.0)
        y = y + yA[i0 * W:(i0 + HC) * W, :]
        y = y * sb2_ref[0:1, :] + sb2_ref[1:2, :]
        y = 0.5 * y * (1.0 + lax.erf(y * INV_SQRT2))
        o_ref[0, i0 * W:(i0 + HC) * W, :] = y.astype(o_ref.dtype)


def _regroup(w, scale):
    C = w.shape[-1]
    return jnp.transpose(w * scale[None, None, None, :],
                         (0, 2, 1, 3)).reshape(3 * C, 3 * C)


def kernel(x, conv_w, conv_b, conv_bn_gamma, conv_bn_beta, conv_bn_mean,
           conv_bn_var, d3_w, d3_b, d3_bn_gamma, d3_bn_beta, d3_bn_mean,
           d3_bn_var, d1_w, d1_b, d1_bn_gamma, d1_bn_beta, d1_bn_mean,
           d1_bn_var, dd3_w, dd3_b, dd3_bn_gamma, dd3_bn_beta, dd3_bn_mean,
           dd3_bn_var, dd5_w, dd5_b, dd5_bn_gamma, dd5_bn_beta, dd5_bn_mean,
           dd5_bn_var, c1_w, c1_b, c1_bn_gamma, c1_bn_beta, c1_bn_mean,
           c1_bn_var, out_bn_gamma, out_bn_beta, out_bn_mean, out_bn_var):
    B, Cin, H, W = x.shape
    C = conv_b.shape[0]
    bf16 = jnp.bfloat16

    sA, bA = _fold_bn(conv_b, conv_bn_gamma, conv_bn_beta, conv_bn_mean,
                      conv_bn_var)
    sB, bB = _fold_bn(d3_b, d3_bn_gamma, d3_bn_beta, d3_bn_mean, d3_bn_var)
    s1d, b1d = _fold_bn(d1_b, d1_bn_gamma, d1_bn_beta, d1_bn_mean, d1_bn_var)
    s3d, b3d = _fold_bn(dd3_b, dd3_bn_gamma, dd3_bn_beta, dd3_bn_mean,
                        dd3_bn_var)
    s5d, b5d = _fold_bn(dd5_b, dd5_bn_gamma, dd5_bn_beta, dd5_bn_mean,
                        dd5_bn_var)
    s1, b1 = _fold_bn(c1_b, c1_bn_gamma, c1_bn_beta, c1_bn_mean, c1_bn_var)
    s2 = out_bn_gamma / jnp.sqrt(out_bn_var + EPS)
    b2 = out_bn_beta - out_bn_mean * s2

    wA = conv_w * sA[None, :]
    w9 = _regroup(d3_w, sB).astype(bf16)
    w3 = jnp.stack([_regroup(d1_w, s1d), _regroup(dd3_w, s3d),
                    _regroup(dd5_w, s5d)]).astype(bf16)
    b3 = jnp.stack([b1d, b3d, b5d])
    w1 = (c1_w.reshape(3, C, C) * s1[None, None, :]).astype(bf16)
    sb2 = jnp.stack([s2, b2])

    return _forward(x, wA, bA.reshape(1, C), w9, bB.reshape(1, C), w3, b3,
                    w1, b1.reshape(1, C), sb2, H=H, W=W, C=C, Ca=Cin)


def _forward(x, wA, bA, w9, bB, w3, b3, w1, b1, sb2, *, H, W, C, Ca):
    B = x.shape[0]
    xp = x.reshape(B, Ca, H * W)
    kern = functools.partial(_fused_kernel, H=H, W=W, C=C, Ca=Ca)
    out = pl.pallas_call(
        kern,
        out_shape=jax.ShapeDtypeStruct((B, H * W, C), jnp.float32),
        grid=(B,),
        in_specs=[
            pl.BlockSpec((1, Ca, H * W), lambda b: (b, 0, 0)),
            pl.BlockSpec((Ca, C), lambda b: (0, 0)),
            pl.BlockSpec((1, C), lambda b: (0, 0)),
            pl.BlockSpec((3 * C, 3 * C), lambda b: (0, 0)),
            pl.BlockSpec((1, C), lambda b: (0, 0)),
            pl.BlockSpec((3, 3 * C, 3 * C), lambda b: (0, 0, 0)),
            pl.BlockSpec((3, C), lambda b: (0, 0)),
            pl.BlockSpec((3, C, C), lambda b: (0, 0, 0)),
            pl.BlockSpec((1, C), lambda b: (0, 0)),
            pl.BlockSpec((2, C), lambda b: (0, 0)),
        ],
        out_specs=pl.BlockSpec((1, H * W, C), lambda b: (b, 0, 0)),
        compiler_params=pltpu.CompilerParams(
            dimension_semantics=("parallel",),
            vmem_limit_bytes=60 * 1024 * 1024),
    )(xp, wA, bA, w9, bB, w3, b3, w1, b1, sb2)

    return jnp.transpose(out.reshape(B, H, W, C), (0, 3, 1, 2))


# speedup vs baseline: 1.1128x; 1.0042x over previous
import functools

import jax
import jax.numpy as jnp
from jax import lax
from jax.experimental import pallas as pl
from jax.experimental.pallas import tpu as pltpu

EPS = 1e-5
INV_SQRT2 = 0.7071067811865476
HC = 16


def _fold_bn(conv_bias, gamma, beta, mean, var):
    scale = gamma / jnp.sqrt(var + EPS)
    bias = beta + (conv_bias - mean) * scale
    return scale, bias


def _conv_rows(src, r0, wc, dil, wg, C, W):
    taps = [src[r0 + (ky - 1) * dil:r0 + (ky - 1) * dil + HC, :, :]
            .reshape(HC * wc, C) for ky in range(3)]
    xcol = jnp.concatenate(taps, axis=-1)
    u = jnp.dot(xcol, wg, preferred_element_type=jnp.float32)
    u3 = u.reshape(HC, wc, 3 * C)
    v = (u3[:, 8 - dil:8 - dil + W, 0:C]
         + u3[:, 8:8 + W, C:2 * C]
         + u3[:, 8 + dil:8 + dil + W, 2 * C:3 * C])
    return v.reshape(HC * W, C)


def _fused_kernel(xp_ref, wA_ref, bA_ref, w9_ref, bB_ref, w3_ref, b3_ref,
                  w1_ref, b1_ref, sb2_ref, o_ref, *, H, W, C, Ca):
    f32 = jnp.float32
    bf16 = jnp.bfloat16
    WF = W + 16

    yA = lax.dot_general(xp_ref[0], wA_ref[...], (((0,), (0,)), ((), ())),
                         preferred_element_type=f32)
    yA = jnp.maximum(yA + bA_ref[...], 0.0)
    y_pad = jnp.pad(yA.astype(bf16).reshape(H, W, C),
                    ((9, 9), (8, 8), (0, 0)))

    chunks = []
    for h0 in range(0, H, HC):
        v = _conv_rows(y_pad, h0 + 9, WF, 1, w9_ref[...], C, W)
        z = jnp.maximum(v + bB_ref[...], 0.0)
        chunks.append(z.astype(bf16).reshape(HC, W, C))
    xb = jnp.pad(jnp.concatenate(chunks, axis=0),
                 ((8, 8), (8, 8), (0, 0)))

    for i0 in range(0, H, HC):
        acc = jnp.zeros((HC * W, C), f32)
        for bi, dil in enumerate((1, 3, 5)):
            v = _conv_rows(xb, i0 + 8, WF, dil, w3_ref[bi], C, W)
            zb = jnp.maximum(v + b3_ref[bi:bi + 1, :], 0.0)
            acc = acc + jnp.dot(zb.astype(bf16), w1_ref[bi],
                                preferred_element_type=f32)
        y = jnp.maximum(acc + b1_ref[...], 0.0)
        y = y + yA[i0 * W:(i0 + HC) * W, :]
        y = y * sb2_ref[0:1, :] + sb2_ref[1:2, :]
        y = 0.5 * y * (1.0 + lax.erf(y * INV_SQRT2))
        o_ref[0, i0 * W:(i0 + HC) * W, :] = y.astype(o_ref.dtype)


def _regroup(w, scale):
    C = w.shape[-1]
    return jnp.transpose(w * scale[None, None, None, :],
                         (0, 2, 1, 3)).reshape(3 * C, 3 * C)


def kernel(x, conv_w, conv_b, conv_bn_gamma, conv_bn_beta, conv_bn_mean,
           conv_bn_var, d3_w, d3_b, d3_bn_gamma, d3_bn_beta, d3_bn_mean,
           d3_bn_var, d1_w, d1_b, d1_bn_gamma, d1_bn_beta, d1_bn_mean,
           d1_bn_var, dd3_w, dd3_b, dd3_bn_gamma, dd3_bn_beta, dd3_bn_mean,
           dd3_bn_var, dd5_w, dd5_b, dd5_bn_gamma, dd5_bn_beta, dd5_bn_mean,
           dd5_bn_var, c1_w, c1_b, c1_bn_gamma, c1_bn_beta, c1_bn_mean,
           c1_bn_var, out_bn_gamma, out_bn_beta, out_bn_mean, out_bn_var):
    B, Cin, H, W = x.shape
    C = conv_b.shape[0]
    bf16 = jnp.bfloat16

    sA, bA = _fold_bn(conv_b, conv_bn_gamma, conv_bn_beta, conv_bn_mean,
                      conv_bn_var)
    sB, bB = _fold_bn(d3_b, d3_bn_gamma, d3_bn_beta, d3_bn_mean, d3_bn_var)
    s1d, b1d = _fold_bn(d1_b, d1_bn_gamma, d1_bn_beta, d1_bn_mean, d1_bn_var)
    s3d, b3d = _fold_bn(dd3_b, dd3_bn_gamma, dd3_bn_beta, dd3_bn_mean,
                        dd3_bn_var)
    s5d, b5d = _fold_bn(dd5_b, dd5_bn_gamma, dd5_bn_beta, dd5_bn_mean,
                        dd5_bn_var)
    s1, b1 = _fold_bn(c1_b, c1_bn_gamma, c1_bn_beta, c1_bn_mean, c1_bn_var)
    s2 = out_bn_gamma / jnp.sqrt(out_bn_var + EPS)
    b2 = out_bn_beta - out_bn_mean * s2

    wA = conv_w * sA[None, :]
    w9 = _regroup(d3_w, sB).astype(bf16)
    w3 = jnp.stack([_regroup(d1_w, s1d), _regroup(dd3_w, s3d),
                    _regroup(dd5_w, s5d)]).astype(bf16)
    b3 = jnp.stack([b1d, b3d, b5d])
    w1 = (c1_w.reshape(3, C, C) * s1[None, None, :]).astype(bf16)
    sb2 = jnp.stack([s2, b2])

    return _forward(x, wA, bA.reshape(1, C), w9, bB.reshape(1, C), w3, b3,
                    w1, b1.reshape(1, C), sb2, H=H, W=W, C=C, Ca=Cin)


def _forward(x, wA, bA, w9, bB, w3, b3, w1, b1, sb2, *, H, W, C, Ca):
    B = x.shape[0]
    xp = x.reshape(B, Ca, H * W)
    kern = functools.partial(_fused_kernel, H=H, W=W, C=C, Ca=Ca)
    out = pl.pallas_call(
        kern,
        out_shape=jax.ShapeDtypeStruct((B, H * W, C), jnp.float32),
        grid=(B,),
        in_specs=[
            pl.BlockSpec((1, Ca, H * W), lambda b: (b, 0, 0)),
            pl.BlockSpec((Ca, C), lambda b: (0, 0)),
            pl.BlockSpec((1, C), lambda b: (0, 0)),
            pl.BlockSpec((3 * C, 3 * C), lambda b: (0, 0)),
            pl.BlockSpec((1, C), lambda b: (0, 0)),
            pl.BlockSpec((3, 3 * C, 3 * C), lambda b: (0, 0, 0)),
            pl.BlockSpec((3, C), lambda b: (0, 0)),
            pl.BlockSpec((3, C, C), lambda b: (0, 0, 0)),
            pl.BlockSpec((1, C), lambda b: (0, 0)),
            pl.BlockSpec((2, C), lambda b: (0, 0)),
        ],
        out_specs=pl.BlockSpec((1, H * W, C), lambda b: (b, 0, 0)),
        compiler_params=pltpu.CompilerParams(
            dimension_semantics=("parallel",),
            vmem_limit_bytes=60 * 1024 * 1024),
    )(xp, wA, bA, w9, bB, w3, b3, w1, b1, sb2)

    return jnp.transpose(out.reshape(B, H, W, C), (0, 3, 1, 2))
